# baseline (speedup 1.0000x reference)
import functools

import jax
import jax.numpy as jnp
from jax import lax
from jax.experimental import pallas as pl
from jax.experimental.pallas import tpu as pltpu
from jax.experimental.pallas import tpu_sc as plsc

N = 100000
NPAD = 100352
C_IN = 16
C_OUT = 16
N_FORE = 50000
N_MID = N_FORE * 26
N_OUT = N_FORE * 28
EPS = 1e-3
SENT = 0x3FFFFFFF

NT = 16
CHUNK = NPAD // NT
VPC = CHUNK // 16
BINS = 1024
SCT = CHUNK // 128
GC = 3128
GCL = N_FORE - 15 * GC
GPAD = 3200
GNG = GPAD // 128

BK0 = 50544
NSORT = 100608
CB = 1024
NCB = (N_OUT + CB - 1) // CB
FORE_HI = (N_FORE - 1) // CB
BACK_SH = (N_MID + N_FORE - BK0) // CB
BACK_HI = (NSORT - 1) // CB


def _digit(kf, shift):
    ki = plsc.bitcast(kf, jnp.int32)
    return lax.shift_right_logical(ki, shift) & (BINS - 1)


def _sc_body(sig_hbm, feat_hbm, out_hbm,
             key_a, idx_a, key_b, idx_b, ghist,
             kbuf, vbuf, dbuf, hist, gh_all, gidx, gbuf,
             sem_g):
    core = lax.axis_index("c")
    si = lax.axis_index("s")

    @pl.when(core == 0)
    def _():
        cnt0, _ = plsc.scan_count(jnp.zeros((16,), jnp.int32))
        adj = jnp.min(cnt0)
        base = pl.multiple_of(si * CHUNK, 8)

        for p in range(3):
            shift = 10 * p
            src_k, src_v, dst_k, dst_v = [
                (sig_hbm, None, key_a, idx_a),
                (key_a, idx_a, key_b, idx_b),
                (key_b, idx_b, key_a, idx_a),
            ][p]
            pltpu.sync_copy(src_k.at[pl.ds(base, CHUNK)], kbuf)
            if src_v is None:
                @pl.loop(0, VPC)
                def _(j):
                    pos = base + j * 16 + lax.iota(jnp.int32, 16)
                    ki = 0x3F800000 - plsc.bitcast(
                        kbuf[pl.ds(j * 16, 16)], jnp.int32)
                    ki = jnp.where(pos < N, ki, SENT)
                    kbuf[pl.ds(j * 16, 16)] = plsc.bitcast(ki, jnp.float32)
                    vbuf[pl.ds(j * 16, 16)] = jnp.where(pos < N, pos, 0)
            else:
                pltpu.sync_copy(src_v.at[pl.ds(base, CHUNK)], vbuf)

            @pl.loop(0, BINS // 16)
            def _(g):
                hist[pl.ds(g * 16, 16)] = jnp.zeros((16,), jnp.int32)

            @pl.loop(0, VPC)
            def _(j):
                d = _digit(kbuf[pl.ds(j * 16, 16)], shift)
                cnt, last = plsc.scan_count(d)
                plsc.addupdate_scatter(hist, [d], cnt - adj + 1, mask=last)

            pltpu.sync_copy(hist, ghist.at[si])
            plsc.subcore_barrier()
            pltpu.sync_copy(ghist, gh_all)

            def grp(g, s_carry):
                tot = jnp.zeros((16,), jnp.int32)
                part = jnp.zeros((16,), jnp.int32)
                for t in range(NT):
                    row = gh_all[t, pl.ds(g * 16, 16)]
                    part = part + jnp.where(t < si, row, 0)
                    tot = tot + row
                excl = plsc.cumsum(tot) - tot + s_carry
                hist[pl.ds(g * 16, 16)] = excl + part
                return s_carry + jnp.sum(tot)

            lax.fori_loop(0, BINS // 16, grp, jnp.int32(0))

            @pl.loop(0, VPC)
            def _(j):
                d = _digit(kbuf[pl.ds(j * 16, 16)], shift)
                cnt, last = plsc.scan_count(d)
                bofs = plsc.load_gather(hist, [d])
                dbuf[lax.div(j, 8), pl.ds(lax.rem(j, 8) * 16, 16)] = (
                    bofs + cnt - adj)
                plsc.addupdate_scatter(hist, [d], cnt - adj + 1, mask=last)

            for i in range(SCT):
                pltpu.make_async_copy(
                    kbuf.at[pl.ds(i * 128, 128)], dst_k.at[dbuf.at[i]],
                    sem_g).start()
                pltpu.make_async_copy(
                    vbuf.at[pl.ds(i * 128, 128)], dst_v.at[dbuf.at[i]],
                    sem_g).start()
            for i in range(SCT):
                pltpu.make_async_copy(
                    kbuf.at[pl.ds(i * 128, 128)], dst_k.at[dbuf.at[i]],
                    sem_g).wait()
                pltpu.make_async_copy(
                    vbuf.at[pl.ds(i * 128, 128)], dst_v.at[dbuf.at[i]],
                    sem_g).wait()
            plsc.subcore_barrier()

        for cc in range(2):
            sstart = pl.multiple_of(cc * N_FORE + si * GC, 8)
            ostart = pl.multiple_of(cc * (BK0 - N_FORE) + sstart, 8)
            pltpu.sync_copy(idx_a.at[pl.ds(sstart, GPAD)], gidx)
            for gj in range(GNG):
                pltpu.make_async_copy(
                    feat_hbm.at[gidx.at[pl.ds(gj * 128, 128)]],
                    gbuf.at[pl.ds(gj * 128, 128)], sem_g).start()
            for gj in range(GNG):
                pltpu.make_async_copy(
                    feat_hbm.at[gidx.at[pl.ds(gj * 128, 128)]],
                    gbuf.at[pl.ds(gj * 128, 128)], sem_g).wait()

            @pl.when(si < 15)
            def _():
                pltpu.sync_copy(gbuf.at[pl.ds(0, GC)],
                                out_hbm.at[pl.ds(ostart, GC)])

            @pl.when(si == 15)
            def _():
                pltpu.sync_copy(gbuf.at[pl.ds(0, GCL)],
                                out_hbm.at[pl.ds(ostart, GCL)])


_sc_sort_gather = functools.partial(
    pl.kernel,
    out_type=jax.ShapeDtypeStruct((NSORT, C_IN), jnp.float32),
    mesh=plsc.VectorSubcoreMesh(core_axis_name="c", subcore_axis_name="s"),
    compiler_params=pltpu.CompilerParams(needs_layout_passes=False,
                                         use_tc_tiling_on_sc=False),
    scratch_types=[
        pltpu.VMEM_SHARED((NPAD,), jnp.float32),
        pltpu.VMEM_SHARED((NPAD,), jnp.int32),
        pltpu.VMEM_SHARED((NPAD,), jnp.float32),
        pltpu.VMEM_SHARED((NPAD,), jnp.int32),
        pltpu.VMEM_SHARED((NT, BINS), jnp.int32),
        pltpu.VMEM((CHUNK,), jnp.float32),
        pltpu.VMEM((CHUNK,), jnp.int32),
        pltpu.VMEM((SCT, 128), jnp.int32),
        pltpu.VMEM((BINS,), jnp.int32),
        pltpu.VMEM((NT, BINS), jnp.int32),
        pltpu.VMEM((GPAD,), jnp.int32),
        pltpu.VMEM((GPAD, C_IN), jnp.float32),
        pltpu.SemaphoreType.DMA,
    ])(_sc_body)


def _asm_body(fore_ref, back_ref, wt_ref, beta_ref, o_ref):
    i = pl.program_id(0)
    col = i * CB + lax.broadcasted_iota(jnp.int32, (CB, 1), 0)
    is_fore = col < N_FORE
    is_back = col >= N_FORE + N_MID
    x = jnp.where(is_fore, fore_ref[...], back_ref[...])
    y = lax.dot_general(wt_ref[...], x, (((0,), (1,)), ((), ())),
                        precision=lax.Precision.HIGHEST,
                        preferred_element_type=jnp.float32)
    y = jnp.maximum(y + beta_ref[...], 0.0)
    o_ref[...] = jnp.where((is_fore | is_back).reshape(1, CB), y, 0.0)


def _assemble(sorted_rows, w_eff, beta):
    return pl.pallas_call(
        _asm_body,
        grid=(NCB,),
        in_specs=[
            pl.BlockSpec((CB, C_IN), lambda i: (jnp.minimum(i, FORE_HI), 0)),
            pl.BlockSpec((CB, C_IN),
                         lambda i: (jnp.clip(i - BACK_SH, 0, BACK_HI), 0)),
            pl.BlockSpec((C_IN, C_OUT), lambda i: (0, 0)),
            pl.BlockSpec((C_OUT, 1), lambda i: (0, 0)),
        ],
        out_specs=pl.BlockSpec((C_OUT, CB), lambda i: (0, i)),
        out_shape=jax.ShapeDtypeStruct((C_OUT, N_OUT), jnp.float32),
    )(sorted_rows, sorted_rows, w_eff, beta)


def kernel(features, indices, W_imp, W_conv, gamma, beta):
    del indices
    imp = features @ W_imp
    sig = jax.nn.sigmoid(imp[:, -1])
    sig_pad = jnp.pad(sig, (0, NPAD - N))

    sorted_rows = _sc_sort_gather(sig_pad, features)
    w_eff = W_conv * (gamma / jnp.sqrt(1.0 + EPS)).reshape(1, C_OUT)
    out_t = _assemble(sorted_rows, w_eff, beta.reshape(C_OUT, 1))
    return out_t.T


# speedup vs baseline: 7.9294x; 7.9294x over previous
import functools

import jax
import jax.numpy as jnp
from jax import lax
from jax.experimental import pallas as pl
from jax.experimental.pallas import tpu as pltpu
from jax.experimental.pallas import tpu_sc as plsc

N = 100000
NPAD = 100352
C_IN = 16
C_OUT = 16
N_FORE = 50000
N_MID = N_FORE * 26
N_OUT = N_FORE * 28
EPS = 1e-3
SENT = 0x3FFFFFFF

NT = 16
CHUNK = NPAD // NT
VPC = CHUNK // 16
BINS = 1024
SCT = CHUNK // 128
GC = 3128
GCL = N_FORE - 15 * GC
GPAD = 3200
GNG = GPAD // 128

BK0 = 50544
NSORT = 100608
CB = 1024
NCB = (N_OUT + CB - 1) // CB
FORE_HI = (N_FORE - 1) // CB
BACK_SH = (N_MID + N_FORE - BK0) // CB
BACK_HI = (NSORT - 1) // CB

GP2 = 1600
HALVES = ((1568, 1560), (1544, 1536))


def _digit(kf, shift):
    ki = plsc.bitcast(kf, jnp.int32)
    return lax.shift_right_logical(ki, shift) & (BINS - 1)


def _sc_body(sig_hbm, featt_hbm, out_hbm,
             key_a, idx_a, key_b, idx_b, ghist,
             kbuf, vbuf, dbuf, hist, gh_all, gidx, idxp, gplanes,
             sem_g):
    core = lax.axis_index("c")
    si = lax.axis_index("s")

    @pl.when(core == 0)
    def _():
        cnt0, _ = plsc.scan_count(jnp.zeros((16,), jnp.int32))
        adj = jnp.min(cnt0)
        base = pl.multiple_of(si * CHUNK, 8)

        for p in range(3):
            shift = 10 * p
            src_k, src_v, dst_k, dst_v = [
                (sig_hbm, None, key_a, idx_a),
                (key_a, idx_a, key_b, idx_b),
                (key_b, idx_b, key_a, idx_a),
            ][p]
            pltpu.sync_copy(src_k.at[pl.ds(base, CHUNK)], kbuf)
            if src_v is None:
                @pl.loop(0, VPC)
                def _(j):
                    pos = base + j * 16 + lax.iota(jnp.int32, 16)
                    ki = 0x3F800000 - plsc.bitcast(
                        kbuf[pl.ds(j * 16, 16)], jnp.int32)
                    ki = jnp.where(pos < N, ki, SENT)
                    kbuf[pl.ds(j * 16, 16)] = plsc.bitcast(ki, jnp.float32)
                    vbuf[pl.ds(j * 16, 16)] = jnp.where(pos < N, pos, 0)
            else:
                pltpu.sync_copy(src_v.at[pl.ds(base, CHUNK)], vbuf)

            @pl.loop(0, BINS // 16)
            def _(g):
                hist[pl.ds(g * 16, 16)] = jnp.zeros((16,), jnp.int32)

            @pl.loop(0, VPC)
            def _(j):
                d = _digit(kbuf[pl.ds(j * 16, 16)], shift)
                cnt, last = plsc.scan_count(d)
                plsc.addupdate_scatter(hist, [d], cnt - adj + 1, mask=last)

            pltpu.sync_copy(hist, ghist.at[si])
            plsc.subcore_barrier()
            pltpu.sync_copy(ghist, gh_all)

            def grp(g, s_carry):
                tot = jnp.zeros((16,), jnp.int32)
                part = jnp.zeros((16,), jnp.int32)
                for t in range(NT):
                    row = gh_all[t, pl.ds(g * 16, 16)]
                    part = part + jnp.where(t < si, row, 0)
                    tot = tot + row
                excl = plsc.cumsum(tot) - tot + s_carry
                hist[pl.ds(g * 16, 16)] = excl + part
                return s_carry + jnp.sum(tot)

            lax.fori_loop(0, BINS // 16, grp, jnp.int32(0))

            @pl.loop(0, VPC)
            def _(j):
                d = _digit(kbuf[pl.ds(j * 16, 16)], shift)
                cnt, last = plsc.scan_count(d)
                bofs = plsc.load_gather(hist, [d])
                dbuf[lax.div(j, 8), pl.ds(lax.rem(j, 8) * 16, 16)] = (
                    bofs + cnt - adj)
                plsc.addupdate_scatter(hist, [d], cnt - adj + 1, mask=last)

            for i in range(SCT):
                pltpu.make_async_copy(
                    kbuf.at[pl.ds(i * 128, 128)], dst_k.at[dbuf.at[i]],
                    sem_g).start()
                pltpu.make_async_copy(
                    vbuf.at[pl.ds(i * 128, 128)], dst_v.at[dbuf.at[i]],
                    sem_g).start()
            for i in range(SCT):
                pltpu.make_async_copy(
                    kbuf.at[pl.ds(i * 128, 128)], dst_k.at[dbuf.at[i]],
                    sem_g).wait()
                pltpu.make_async_copy(
                    vbuf.at[pl.ds(i * 128, 128)], dst_v.at[dbuf.at[i]],
                    sem_g).wait()
            plsc.subcore_barrier()

        for cc in range(2):
            for hh in range(2):
                off = (0 if hh == 0 else HALVES[0][0],
                       0 if hh == 0 else HALVES[1][0])
                sstart = pl.multiple_of(
                    cc * N_FORE + si * GC
                    + jnp.where(si < 15, off[0], off[1]), 8)
                ostart = pl.multiple_of(cc * (BK0 - N_FORE) + sstart, 8)
                pltpu.sync_copy(idx_a.at[pl.ds(sstart, GP2)], gidx)
                for c in range(C_IN):
                    @pl.loop(0, GP2 // 16)
                    def _(j):
                        idxp[c, pl.ds(j * 16, 16)] = (
                            gidx[pl.ds(j * 16, 16)] + c * N)
                for c in range(C_IN):
                    pltpu.make_async_copy(
                        featt_hbm.at[idxp.at[c]], gplanes.at[c],
                        sem_g).start()
                for c in range(C_IN):
                    pltpu.make_async_copy(
                        featt_hbm.at[idxp.at[c]], gplanes.at[c],
                        sem_g).wait()
                for c in range(C_IN):
                    @pl.when(si < 15)
                    def _():
                        sz = HALVES[0][hh]
                        pltpu.sync_copy(
                            gplanes.at[c, pl.ds(0, sz)],
                            out_hbm.at[c, pl.ds(ostart, sz)])

                    @pl.when(si == 15)
                    def _():
                        sz = HALVES[1][hh]
                        pltpu.sync_copy(
                            gplanes.at[c, pl.ds(0, sz)],
                            out_hbm.at[c, pl.ds(ostart, sz)])


_sc_sort_gather = functools.partial(
    pl.kernel,
    out_type=jax.ShapeDtypeStruct((C_IN, NSORT), jnp.float32),
    mesh=plsc.VectorSubcoreMesh(core_axis_name="c", subcore_axis_name="s"),
    compiler_params=pltpu.CompilerParams(needs_layout_passes=False,
                                         use_tc_tiling_on_sc=False),
    scratch_types=[
        pltpu.VMEM_SHARED((NPAD,), jnp.float32),
        pltpu.VMEM_SHARED((NPAD,), jnp.int32),
        pltpu.VMEM_SHARED((NPAD,), jnp.float32),
        pltpu.VMEM_SHARED((NPAD,), jnp.int32),
        pltpu.VMEM_SHARED((NT, BINS), jnp.int32),
        pltpu.VMEM((CHUNK,), jnp.float32),
        pltpu.VMEM((CHUNK,), jnp.int32),
        pltpu.VMEM((SCT, 128), jnp.int32),
        pltpu.VMEM((BINS,), jnp.int32),
        pltpu.VMEM((NT, BINS), jnp.int32),
        pltpu.VMEM((GP2,), jnp.int32),
        pltpu.VMEM((C_IN, GP2), jnp.int32),
        pltpu.VMEM((C_IN, GP2), jnp.float32),
        pltpu.SemaphoreType.DMA,
    ])(_sc_body)


def _asm_body(fore_ref, back_ref, wt_ref, beta_ref, o_ref):
    i = pl.program_id(0)
    col = i * CB + lax.broadcasted_iota(jnp.int32, (1, CB), 1)
    is_fore = col < N_FORE
    is_back = col >= N_FORE + N_MID
    x = jnp.where(is_fore, fore_ref[...], back_ref[...])
    y = lax.dot_general(wt_ref[...], x, (((0,), (0,)), ((), ())),
                        precision=lax.Precision.HIGHEST,
                        preferred_element_type=jnp.float32)
    y = jnp.maximum(y + beta_ref[...], 0.0)
    o_ref[...] = jnp.where(is_fore | is_back, y, 0.0)


def _assemble(sorted_t, w_eff, beta):
    return pl.pallas_call(
        _asm_body,
        grid=(NCB,),
        in_specs=[
            pl.BlockSpec((C_IN, CB), lambda i: (0, jnp.minimum(i, FORE_HI))),
            pl.BlockSpec((C_IN, CB),
                         lambda i: (0, jnp.clip(i - BACK_SH, 0, BACK_HI))),
            pl.BlockSpec((C_IN, C_OUT), lambda i: (0, 0)),
            pl.BlockSpec((C_OUT, 1), lambda i: (0, 0)),
        ],
        out_specs=pl.BlockSpec((C_OUT, CB), lambda i: (0, i)),
        out_shape=jax.ShapeDtypeStruct((C_OUT, N_OUT), jnp.float32),
    )(sorted_t, sorted_t, w_eff, beta)


def kernel(features, indices, W_imp, W_conv, gamma, beta):
    del indices
    imp = features @ W_imp
    sig = jax.nn.sigmoid(imp[:, -1])
    sig_pad = jnp.pad(sig, (0, NPAD - N))

    featt_flat = features.T.reshape(C_IN * N)
    sorted_t = _sc_sort_gather(sig_pad, featt_flat)
    w_eff = W_conv * (gamma / jnp.sqrt(1.0 + EPS)).reshape(1, C_OUT)
    out_t = _assemble(sorted_t, w_eff, beta.reshape(C_OUT, 1))
    return out_t.T


# speedup vs baseline: 15.1438x; 1.9098x over previous
import functools

import jax
import jax.numpy as jnp
from jax import lax
from jax.experimental import pallas as pl
from jax.experimental.pallas import tpu as pltpu
from jax.experimental.pallas import tpu_sc as plsc

N = 100000
NPAD = 100352
C_IN = 16
C_OUT = 16
N_FORE = 50000
N_MID = N_FORE * 26
N_OUT = N_FORE * 28
EPS = 1e-3
SENT = 0x3FFFFFFF

NT = 16
CHUNK = NPAD // NT
VPC = CHUNK // 16
BINS = 1024
SCT = CHUNK // 128
GC = 3128
GCL = N_FORE - 15 * GC
GPAD = 3200
GNG = GPAD // 128

BK0 = 51568
NSORT = 102400
CB = 4096
NCB = (N_OUT + CB - 1) // CB
FORE_HI = (N_FORE - 1) // CB
BACK_SH = (N_MID + N_FORE - BK0) // CB
BACK_HI = (NSORT - 1) // CB

GP2 = 1600
HALVES = ((1568, 1560), (1544, 1536))


def _digit(kf, shift):
    ki = plsc.bitcast(kf, jnp.int32)
    return lax.shift_right_logical(ki, shift) & (BINS - 1)


def _sc_body(sig_hbm, featt_hbm, out_hbm,
             key_a, idx_a, key_b, idx_b, ghist,
             kbuf, vbuf, dbuf, hist, gh_all, gidx, idxp, gplanes,
             sem_g):
    core = lax.axis_index("c")
    si = lax.axis_index("s")

    @pl.when(core == 0)
    def _():
        cnt0, _ = plsc.scan_count(jnp.zeros((16,), jnp.int32))
        adj = jnp.min(cnt0)
        base = pl.multiple_of(si * CHUNK, 8)

        for p in range(3):
            shift = 10 * p
            src_k, src_v, dst_k, dst_v = [
                (sig_hbm, None, key_a, idx_a),
                (key_a, idx_a, key_b, idx_b),
                (key_b, idx_b, key_a, idx_a),
            ][p]
            pltpu.sync_copy(src_k.at[pl.ds(base, CHUNK)], kbuf)
            if src_v is None:
                @pl.loop(0, VPC)
                def _(j):
                    pos = base + j * 16 + lax.iota(jnp.int32, 16)
                    ki = 0x3F800000 - plsc.bitcast(
                        kbuf[pl.ds(j * 16, 16)], jnp.int32)
                    ki = jnp.where(pos < N, ki, SENT)
                    kbuf[pl.ds(j * 16, 16)] = plsc.bitcast(ki, jnp.float32)
                    vbuf[pl.ds(j * 16, 16)] = jnp.where(pos < N, pos, 0)
            else:
                pltpu.sync_copy(src_v.at[pl.ds(base, CHUNK)], vbuf)

            @pl.loop(0, BINS // 16)
            def _(g):
                hist[pl.ds(g * 16, 16)] = jnp.zeros((16,), jnp.int32)

            @pl.loop(0, VPC)
            def _(j):
                d = _digit(kbuf[pl.ds(j * 16, 16)], shift)
                cnt, last = plsc.scan_count(d)
                plsc.addupdate_scatter(hist, [d], cnt - adj + 1, mask=last)

            pltpu.sync_copy(hist, ghist.at[si])
            plsc.subcore_barrier()
            pltpu.sync_copy(ghist, gh_all)

            def grp(g, s_carry):
                tot = jnp.zeros((16,), jnp.int32)
                part = jnp.zeros((16,), jnp.int32)
                for t in range(NT):
                    row = gh_all[t, pl.ds(g * 16, 16)]
                    part = part + jnp.where(t < si, row, 0)
                    tot = tot + row
                excl = plsc.cumsum(tot) - tot + s_carry
                hist[pl.ds(g * 16, 16)] = excl + part
                return s_carry + jnp.sum(tot)

            lax.fori_loop(0, BINS // 16, grp, jnp.int32(0))

            @pl.loop(0, VPC)
            def _(j):
                d = _digit(kbuf[pl.ds(j * 16, 16)], shift)
                cnt, last = plsc.scan_count(d)
                bofs = plsc.load_gather(hist, [d])
                dbuf[lax.div(j, 8), pl.ds(lax.rem(j, 8) * 16, 16)] = (
                    bofs + cnt - adj)
                plsc.addupdate_scatter(hist, [d], cnt - adj + 1, mask=last)

            for i in range(SCT):
                pltpu.make_async_copy(
                    kbuf.at[pl.ds(i * 128, 128)], dst_k.at[dbuf.at[i]],
                    sem_g).start()
                pltpu.make_async_copy(
                    vbuf.at[pl.ds(i * 128, 128)], dst_v.at[dbuf.at[i]],
                    sem_g).start()
            for i in range(SCT):
                pltpu.make_async_copy(
                    kbuf.at[pl.ds(i * 128, 128)], dst_k.at[dbuf.at[i]],
                    sem_g).wait()
                pltpu.make_async_copy(
                    vbuf.at[pl.ds(i * 128, 128)], dst_v.at[dbuf.at[i]],
                    sem_g).wait()
            plsc.subcore_barrier()

        for cc in range(2):
            for hh in range(2):
                off = (0 if hh == 0 else HALVES[0][0],
                       0 if hh == 0 else HALVES[1][0])
                sstart = pl.multiple_of(
                    cc * N_FORE + si * GC
                    + jnp.where(si < 15, off[0], off[1]), 8)
                ostart = pl.multiple_of(cc * (BK0 - N_FORE) + sstart, 8)
                pltpu.sync_copy(idx_a.at[pl.ds(sstart, GP2)], gidx)
                for c in range(C_IN):
                    @pl.loop(0, GP2 // 16)
                    def _(j):
                        idxp[c, pl.ds(j * 16, 16)] = (
                            gidx[pl.ds(j * 16, 16)] + c * N)
                for c in range(C_IN):
                    pltpu.make_async_copy(
                        featt_hbm.at[idxp.at[c]], gplanes.at[c],
                        sem_g).start()
                for c in range(C_IN):
                    pltpu.make_async_copy(
                        featt_hbm.at[idxp.at[c]], gplanes.at[c],
                        sem_g).wait()
                for c in range(C_IN):
                    @pl.when(si < 15)
                    def _():
                        sz = HALVES[0][hh]
                        pltpu.sync_copy(
                            gplanes.at[c, pl.ds(0, sz)],
                            out_hbm.at[c, pl.ds(ostart, sz)])

                    @pl.when(si == 15)
                    def _():
                        sz = HALVES[1][hh]
                        pltpu.sync_copy(
                            gplanes.at[c, pl.ds(0, sz)],
                            out_hbm.at[c, pl.ds(ostart, sz)])


_sc_sort_gather = functools.partial(
    pl.kernel,
    out_type=jax.ShapeDtypeStruct((C_IN, NSORT), jnp.float32),
    mesh=plsc.VectorSubcoreMesh(core_axis_name="c", subcore_axis_name="s"),
    compiler_params=pltpu.CompilerParams(needs_layout_passes=False,
                                         use_tc_tiling_on_sc=False),
    scratch_types=[
        pltpu.VMEM_SHARED((NPAD,), jnp.float32),
        pltpu.VMEM_SHARED((NPAD,), jnp.int32),
        pltpu.VMEM_SHARED((NPAD,), jnp.float32),
        pltpu.VMEM_SHARED((NPAD,), jnp.int32),
        pltpu.VMEM_SHARED((NT, BINS), jnp.int32),
        pltpu.VMEM((CHUNK,), jnp.float32),
        pltpu.VMEM((CHUNK,), jnp.int32),
        pltpu.VMEM((SCT, 128), jnp.int32),
        pltpu.VMEM((BINS,), jnp.int32),
        pltpu.VMEM((NT, BINS), jnp.int32),
        pltpu.VMEM((GP2,), jnp.int32),
        pltpu.VMEM((C_IN, GP2), jnp.int32),
        pltpu.VMEM((C_IN, GP2), jnp.float32),
        pltpu.SemaphoreType.DMA,
    ])(_sc_body)


def _asm_body(fore_ref, back_ref, wt_ref, beta_ref, o_ref):
    i = pl.program_id(0)
    col = i * CB + lax.broadcasted_iota(jnp.int32, (1, CB), 1)
    is_fore = col < N_FORE
    is_back = col >= N_FORE + N_MID
    x = jnp.where(is_fore, fore_ref[...], back_ref[...])
    y = lax.dot_general(wt_ref[...], x, (((0,), (0,)), ((), ())),
                        precision=lax.Precision.HIGHEST,
                        preferred_element_type=jnp.float32)
    y = jnp.maximum(y + beta_ref[...], 0.0)
    o_ref[...] = jnp.where(is_fore | is_back, y, 0.0)


def _assemble(sorted_t, w_eff, beta):
    return pl.pallas_call(
        _asm_body,
        grid=(NCB,),
        in_specs=[
            pl.BlockSpec((C_IN, CB), lambda i: (0, jnp.minimum(i, FORE_HI))),
            pl.BlockSpec((C_IN, CB),
                         lambda i: (0, jnp.clip(i - BACK_SH, 0, BACK_HI))),
            pl.BlockSpec((C_IN, C_OUT), lambda i: (0, 0)),
            pl.BlockSpec((C_OUT, 1), lambda i: (0, 0)),
        ],
        out_specs=pl.BlockSpec((C_OUT, CB), lambda i: (0, i)),
        out_shape=jax.ShapeDtypeStruct((C_OUT, N_OUT), jnp.float32),
    )(sorted_t, sorted_t, w_eff, beta)


def kernel(features, indices, W_imp, W_conv, gamma, beta):
    del indices
    imp = features @ W_imp
    sig = jax.nn.sigmoid(imp[:, -1])
    sig_pad = jnp.pad(sig, (0, NPAD - N))

    featt_flat = features.T.reshape(C_IN * N)
    sorted_t = _sc_sort_gather(sig_pad, featt_flat)
    w_eff = W_conv * (gamma / jnp.sqrt(1.0 + EPS)).reshape(1, C_OUT)
    out_t = _assemble(sorted_t, w_eff, beta.reshape(C_OUT, 1))
    return out_t.T
